# Initial kernel scaffold; baseline (speedup 1.0000x reference)
import jax
import jax.numpy as jnp
from jax.experimental import pallas as pl


def kernel(x, tables):
    raise NotImplementedError("write your pallas kernel here")


# speedup vs baseline: 1.0320x; 1.0320x over previous
import functools

import jax
import jax.numpy as jnp
from jax import lax
from jax.experimental import pallas as pl
from jax.experimental.pallas import tpu as pltpu
from jax.experimental.pallas import tpu_sc as plsc

NUM_NUMERICAL = 13
NUM_EMBED = 26
VOCAB = 100000
D = 64
BATCH = 16384
OUT_W = NUM_NUMERICAL + NUM_EMBED * D

NC = 2
NS = 16
NW = NC * NS

IDX_TOTAL = BATCH * NUM_EMBED
IDX_PER_W = IDX_TOTAL // NW
CHUNK = 128
GATHERS_PER_STEP = 4
STEP = CHUNK * GATHERS_PER_STEP
STEPS = IDX_PER_W // STEP

_mesh = plsc.VectorSubcoreMesh(core_axis_name="c", subcore_axis_name="s")


@functools.partial(
    pl.kernel,
    out_type=jax.ShapeDtypeStruct((IDX_TOTAL, D), jnp.float32),
    mesh=_mesh,
    scratch_types=[
        pltpu.VMEM((IDX_PER_W,), jnp.int32),
        pltpu.VMEM((STEP, D), jnp.float32),
        pltpu.SemaphoreType.DMA,
    ],
    compiler_params=pltpu.CompilerParams(use_tc_tiling_on_sc=False),
)
def _sc_gather(tables_hbm, idx_hbm, out_hbm, idx_v, buf_v, sem):
    wid = lax.axis_index("s") * NC + lax.axis_index("c")
    base = wid * IDX_PER_W
    pltpu.sync_copy(idx_hbm.at[pl.ds(base, IDX_PER_W)], idx_v)

    @pl.loop(0, STEPS)
    def _(step):
        off = step * STEP
        copies = []
        for g in range(GATHERS_PER_STEP):
            copies.append(
                pltpu.async_copy(
                    tables_hbm.at[idx_v.at[pl.ds(off + g * CHUNK, CHUNK)]],
                    buf_v.at[pl.ds(g * CHUNK, CHUNK)],
                    sem,
                )
            )
        for c in copies:
            c.wait()
        pltpu.sync_copy(buf_v, out_hbm.at[pl.ds(base + off, STEP)])


_RB = 256


def _concat_body(x_ref, emb_ref, o_ref):
    o_ref[:, :NUM_NUMERICAL] = x_ref[:, :NUM_NUMERICAL]
    o_ref[:, NUM_NUMERICAL:] = emb_ref[...]


_concat = pl.pallas_call(
    _concat_body,
    out_shape=jax.ShapeDtypeStruct((BATCH, OUT_W), jnp.float32),
    grid=(BATCH // _RB,),
    in_specs=[
        pl.BlockSpec((_RB, NUM_NUMERICAL + NUM_EMBED), lambda i: (i, 0)),
        pl.BlockSpec((_RB, NUM_EMBED * D), lambda i: (i, 0)),
    ],
    out_specs=pl.BlockSpec((_RB, OUT_W), lambda i: (i, 0)),
)


def kernel(x, tables):
    tables_flat = tables.reshape(NUM_EMBED * VOCAB, D)
    offs = (jnp.arange(NUM_EMBED, dtype=jnp.int32) * VOCAB)[None, :]
    idx = (x[:, NUM_NUMERICAL:].astype(jnp.int32) + offs).reshape(-1)
    emb = _sc_gather(tables_flat, idx)
    return _concat(x, emb.reshape(BATCH, NUM_EMBED * D))
